# initial kernel scaffold (unmeasured)
import jax
import jax.numpy as jnp
from jax import lax
from jax.experimental import pallas as pl
from jax.experimental.pallas import tpu as pltpu

T = 4096
D = 2048
F = 4096
E_LOCAL = 4
CAP = 1280
BT = 256

_MESH = pl.DeviceIdType.MESH


def _peer():
    return (1 - lax.axis_index("x"), lax.axis_index("y"))


def _neighbor_barrier(peer):
    bar = pltpu.get_barrier_semaphore()
    pl.semaphore_signal(bar, inc=1, device_id=peer, device_id_type=_MESH)
    pl.semaphore_wait(bar, 1)


def _exchange(x_bf, a2d):

    def body(x_ref, a_ref, xo_ref, ao_ref, xs_sem, xr_sem, as_sem, ar_sem):
        peer = _peer()
        _neighbor_barrier(peer)
        rx = pltpu.make_async_remote_copy(
            src_ref=x_ref, dst_ref=xo_ref, send_sem=xs_sem, recv_sem=xr_sem,
            device_id=peer, device_id_type=_MESH,
        )
        ra = pltpu.make_async_remote_copy(
            src_ref=a_ref, dst_ref=ao_ref, send_sem=as_sem, recv_sem=ar_sem,
            device_id=peer, device_id_type=_MESH,
        )
        rx.start()
        ra.start()
        rx.wait()
        ra.wait()

    return pl.pallas_call(
        body,
        out_shape=(
            jax.ShapeDtypeStruct((T, D), jnp.bfloat16),
            jax.ShapeDtypeStruct(a2d.shape, jnp.int32),
        ),
        in_specs=[
            pl.BlockSpec(memory_space=pltpu.VMEM),
            pl.BlockSpec(memory_space=pltpu.VMEM),
        ],
        out_specs=(
            pl.BlockSpec(memory_space=pltpu.VMEM),
            pl.BlockSpec(memory_space=pltpu.VMEM),
        ),
        scratch_shapes=[pltpu.SemaphoreType.DMA] * 4,
        compiler_params=pltpu.CompilerParams(collective_id=0),
    )(x_bf, a2d)


def _moe_ffn(xg, w1, w2):

    def body(x_ref, w1_ref, w2_ref, o_ref):
        h = jnp.dot(x_ref[0], w1_ref[0], preferred_element_type=jnp.float32)
        h = jnp.maximum(h, 0.0).astype(jnp.bfloat16)
        o_ref[0] = jnp.dot(
            h, w2_ref[0], preferred_element_type=jnp.float32
        ).astype(jnp.bfloat16)

    return pl.pallas_call(
        body,
        grid=(E_LOCAL, CAP // BT),
        in_specs=[
            pl.BlockSpec((1, BT, D), lambda e, t: (e, t, 0)),
            pl.BlockSpec((1, D, F), lambda e, t: (e, 0, 0)),
            pl.BlockSpec((1, F, D), lambda e, t: (e, 0, 0)),
        ],
        out_specs=pl.BlockSpec((1, BT, D), lambda e, t: (e, t, 0)),
        out_shape=jax.ShapeDtypeStruct((E_LOCAL, CAP, D), jnp.bfloat16),
    )(xg, w1, w2)


def _combine(mine_bf, theirs_bf):

    def body(mine_ref, theirs_ref, out_ref, recv_ref, s_sem, r_sem):
        peer = _peer()
        _neighbor_barrier(peer)
        rdma = pltpu.make_async_remote_copy(
            src_ref=theirs_ref, dst_ref=recv_ref, send_sem=s_sem,
            recv_sem=r_sem, device_id=peer, device_id_type=_MESH,
        )
        rdma.start()
        rdma.wait()
        out_ref[...] = (
            mine_ref[...].astype(jnp.float32) + recv_ref[...].astype(jnp.float32)
        )

    return pl.pallas_call(
        body,
        out_shape=jax.ShapeDtypeStruct((T, D), jnp.float32),
        in_specs=[
            pl.BlockSpec(memory_space=pltpu.VMEM),
            pl.BlockSpec(memory_space=pltpu.VMEM),
        ],
        out_specs=pl.BlockSpec(memory_space=pltpu.VMEM),
        scratch_shapes=[
            pltpu.VMEM((T, D), jnp.bfloat16),
            pltpu.SemaphoreType.DMA,
            pltpu.SemaphoreType.DMA,
        ],
        compiler_params=pltpu.CompilerParams(collective_id=1),
    )(mine_bf, theirs_bf)


def kernel(x, assign, W1, W2):
    my_x = lax.axis_index("x")
    x_bf = x.astype(jnp.bfloat16)
    a2d = assign.reshape(32, 128)

    x_other, a_other = _exchange(x_bf, a2d)

    x_all = jnp.concatenate([x_bf, x_other], axis=0)
    assign_all = jnp.concatenate([assign, a_other.reshape(-1)])

    e_base = my_x * E_LOCAL
    idx = jnp.stack(
        [
            jnp.nonzero(assign_all == e_base + i, size=CAP, fill_value=2 * T)[0]
            for i in range(E_LOCAL)
        ]
    )
    xg = x_all[idx]

    yg = _moe_ffn(xg, W1.astype(jnp.bfloat16), W2.astype(jnp.bfloat16))

    partial = (
        jnp.zeros((2 * T, D), jnp.bfloat16)
        .at[idx.reshape(-1)]
        .set(yg.reshape(-1, D), mode="drop")
    )

    return _combine(partial[:T], partial[T:])


# baseline (device time: 3427671 ns/iter reference)
import jax
import jax.numpy as jnp
from jax import lax
from jax.experimental import pallas as pl
from jax.experimental.pallas import tpu as pltpu

T = 4096
D = 2048
F = 4096
E_LOCAL = 4
CAP = 1280
BT = 256

_MESH = pl.DeviceIdType.MESH
_VMEM_LIMIT = 60 * 1024 * 1024


def _peer():
    return (1 - lax.axis_index("x"), lax.axis_index("y"))


def _neighbor_barrier(peer):
    bar = pltpu.get_barrier_semaphore()
    pl.semaphore_signal(bar, inc=1, device_id=peer, device_id_type=_MESH)
    pl.semaphore_wait(bar, 1)


def _exchange(x_bf, a2d):

    def body(x_ref, a_ref, xo_ref, ao_ref, xs_sem, xr_sem, as_sem, ar_sem):
        peer = _peer()
        _neighbor_barrier(peer)
        rx = pltpu.make_async_remote_copy(
            src_ref=x_ref, dst_ref=xo_ref, send_sem=xs_sem, recv_sem=xr_sem,
            device_id=peer, device_id_type=_MESH,
        )
        ra = pltpu.make_async_remote_copy(
            src_ref=a_ref, dst_ref=ao_ref, send_sem=as_sem, recv_sem=ar_sem,
            device_id=peer, device_id_type=_MESH,
        )
        rx.start()
        ra.start()
        rx.wait()
        ra.wait()

    return pl.pallas_call(
        body,
        out_shape=(
            jax.ShapeDtypeStruct((T, D), jnp.bfloat16),
            jax.ShapeDtypeStruct(a2d.shape, jnp.int32),
        ),
        in_specs=[
            pl.BlockSpec(memory_space=pltpu.VMEM),
            pl.BlockSpec(memory_space=pltpu.VMEM),
        ],
        out_specs=(
            pl.BlockSpec(memory_space=pltpu.VMEM),
            pl.BlockSpec(memory_space=pltpu.VMEM),
        ),
        scratch_shapes=[pltpu.SemaphoreType.DMA] * 4,
        compiler_params=pltpu.CompilerParams(
            collective_id=0, vmem_limit_bytes=_VMEM_LIMIT
        ),
    )(x_bf, a2d)


FB = 2048
NF = F // FB


def _moe_ffn(xg, w1, w2):

    def body(x_ref, w1_ref, w2_ref, o_ref, acc_ref):
        f = pl.program_id(2)
        h = jnp.dot(x_ref[0], w1_ref[0], preferred_element_type=jnp.float32)
        h = jnp.maximum(h, 0.0).astype(jnp.bfloat16)
        p = jnp.dot(h, w2_ref[0], preferred_element_type=jnp.float32)

        @pl.when(f == 0)
        def _():
            acc_ref[...] = p

        @pl.when(f != 0)
        def _():
            acc_ref[...] += p

        @pl.when(f == NF - 1)
        def _():
            o_ref[0] = acc_ref[...].astype(jnp.bfloat16)

    return pl.pallas_call(
        body,
        grid=(E_LOCAL, CAP // BT, NF),
        in_specs=[
            pl.BlockSpec((1, BT, D), lambda e, t, f: (e, t, 0)),
            pl.BlockSpec((1, D, FB), lambda e, t, f: (e, 0, f)),
            pl.BlockSpec((1, FB, D), lambda e, t, f: (e, f, 0)),
        ],
        out_specs=pl.BlockSpec((1, BT, D), lambda e, t, f: (e, t, 0)),
        out_shape=jax.ShapeDtypeStruct((E_LOCAL, CAP, D), jnp.bfloat16),
        scratch_shapes=[pltpu.VMEM((BT, D), jnp.float32)],
        compiler_params=pltpu.CompilerParams(vmem_limit_bytes=_VMEM_LIMIT),
    )(xg, w1, w2)


def _exchange_back(theirs_bf):

    def body(theirs_ref, recv_ref, s_sem, r_sem):
        peer = _peer()
        _neighbor_barrier(peer)
        rdma = pltpu.make_async_remote_copy(
            src_ref=theirs_ref, dst_ref=recv_ref, send_sem=s_sem,
            recv_sem=r_sem, device_id=peer, device_id_type=_MESH,
        )
        rdma.start()
        rdma.wait()

    return pl.pallas_call(
        body,
        out_shape=jax.ShapeDtypeStruct((T, D), jnp.bfloat16),
        in_specs=[pl.BlockSpec(memory_space=pltpu.VMEM)],
        out_specs=pl.BlockSpec(memory_space=pltpu.VMEM),
        scratch_shapes=[pltpu.SemaphoreType.DMA, pltpu.SemaphoreType.DMA],
        compiler_params=pltpu.CompilerParams(
            collective_id=1, vmem_limit_bytes=_VMEM_LIMIT
        ),
    )(theirs_bf)


def kernel(x, assign, W1, W2):
    my_x = lax.axis_index("x")
    x_bf = x.astype(jnp.bfloat16)
    a2d = assign.reshape(32, 128)

    x_other, a_other = _exchange(x_bf, a2d)

    x_all = jnp.concatenate([x_bf, x_other], axis=0)
    assign_all = jnp.concatenate([assign, a_other.reshape(-1)])

    e_base = my_x * E_LOCAL
    idx = jnp.stack(
        [
            jnp.nonzero(assign_all == e_base + i, size=CAP, fill_value=2 * T)[0]
            for i in range(E_LOCAL)
        ]
    )
    xg = x_all[idx]

    yg = _moe_ffn(xg, W1.astype(jnp.bfloat16), W2.astype(jnp.bfloat16))

    partial = (
        jnp.zeros((2 * T, D), jnp.bfloat16)
        .at[idx.reshape(-1)]
        .set(yg.reshape(-1, D), mode="drop")
    )

    recv = _exchange_back(partial[T:])
    return partial[:T].astype(jnp.float32) + recv.astype(jnp.float32)


# device time: 1189179 ns/iter; 2.8824x vs baseline; 2.8824x over previous
import jax
import jax.numpy as jnp
from jax import lax
from jax.experimental import pallas as pl
from jax.experimental.pallas import tpu as pltpu

T = 4096
D = 2048
F = 4096
E_LOCAL = 4
CAP = 1280
BT = 256
FB = 1024
NF = F // FB
CG = 1024
CS = 1024

_MESH = pl.DeviceIdType.MESH
_VMEM_LIMIT = 60 * 1024 * 1024


def _peer():
    return (1 - lax.axis_index("x"), lax.axis_index("y"))


def _neighbor_barrier(peer):
    bar = pltpu.get_barrier_semaphore()
    pl.semaphore_signal(bar, inc=1, device_id=peer, device_id_type=_MESH)
    pl.semaphore_wait(bar, 1)


def _exchange(x_bf, a2d):

    def body(x_ref, a_ref, xo_ref, ao_ref, xs_sem, xr_sem, as_sem, ar_sem):
        peer = _peer()
        _neighbor_barrier(peer)
        rx = pltpu.make_async_remote_copy(
            src_ref=x_ref, dst_ref=xo_ref, send_sem=xs_sem, recv_sem=xr_sem,
            device_id=peer, device_id_type=_MESH,
        )
        ra = pltpu.make_async_remote_copy(
            src_ref=a_ref, dst_ref=ao_ref, send_sem=as_sem, recv_sem=ar_sem,
            device_id=peer, device_id_type=_MESH,
        )
        rx.start()
        ra.start()
        rx.wait()
        ra.wait()

    return pl.pallas_call(
        body,
        out_shape=(
            jax.ShapeDtypeStruct((T, D), jnp.bfloat16),
            jax.ShapeDtypeStruct(a2d.shape, jnp.int32),
        ),
        in_specs=[
            pl.BlockSpec(memory_space=pltpu.VMEM),
            pl.BlockSpec(memory_space=pltpu.VMEM),
        ],
        out_specs=(
            pl.BlockSpec(memory_space=pltpu.VMEM),
            pl.BlockSpec(memory_space=pltpu.VMEM),
        ),
        scratch_shapes=[pltpu.SemaphoreType.DMA] * 4,
        compiler_params=pltpu.CompilerParams(
            collective_id=0, vmem_limit_bytes=_VMEM_LIMIT
        ),
    )(x_bf, a2d)


def _gather_onehot(idx, x_mine, x_other):

    def body(idx_ref, xm_ref, xo_ref, o_ref):
        e = pl.program_id(0)
        c = pl.program_id(1)
        ids = idx_ref[e][:, None]
        pos = lax.broadcasted_iota(jnp.int32, (CAP, CG), 1) + c * CG
        pm = (ids == pos).astype(jnp.bfloat16)
        po = (ids == pos + T).astype(jnp.bfloat16)
        contrib = jnp.dot(
            pm, xm_ref[...], preferred_element_type=jnp.float32
        ) + jnp.dot(po, xo_ref[...], preferred_element_type=jnp.float32)

        @pl.when(c == 0)
        def _():
            o_ref[0] = contrib.astype(jnp.bfloat16)

        @pl.when(c != 0)
        def _():
            o_ref[0] = o_ref[0] + contrib.astype(jnp.bfloat16)

    return pl.pallas_call(
        body,
        grid=(E_LOCAL, T // CG),
        in_specs=[
            pl.BlockSpec(memory_space=pltpu.VMEM),
            pl.BlockSpec((CG, D), lambda e, c: (c, 0)),
            pl.BlockSpec((CG, D), lambda e, c: (c, 0)),
        ],
        out_specs=pl.BlockSpec((1, CAP, D), lambda e, c: (e, 0, 0)),
        out_shape=jax.ShapeDtypeStruct((E_LOCAL, CAP, D), jnp.bfloat16),
        compiler_params=pltpu.CompilerParams(vmem_limit_bytes=_VMEM_LIMIT),
    )(idx, x_mine, x_other)


def _moe_ffn(xg, w1, w2):

    def body(x_ref, w1_ref, w2_ref, o_ref, w1b, w2b, acc_ref):
        f = pl.program_id(1)
        t = pl.program_id(2)

        @pl.when(t == 0)
        def _():
            w1b[...] = w1_ref[0].astype(jnp.bfloat16)
            w2b[...] = w2_ref[0].astype(jnp.bfloat16)

        h = jnp.dot(x_ref[0], w1b[...], preferred_element_type=jnp.float32)
        h = jnp.maximum(h, 0.0).astype(jnp.bfloat16)
        p = jnp.dot(h, w2b[...], preferred_element_type=jnp.float32)
        sl = pl.ds(t * BT, BT)

        @pl.when(f == 0)
        def _():
            acc_ref[sl, :] = p

        @pl.when(f != 0)
        def _():
            acc_ref[sl, :] += p

        @pl.when(f == NF - 1)
        def _():
            o_ref[0] = acc_ref[sl, :].astype(jnp.bfloat16)

    return pl.pallas_call(
        body,
        grid=(E_LOCAL, NF, CAP // BT),
        in_specs=[
            pl.BlockSpec((1, BT, D), lambda e, f, t: (e, t, 0)),
            pl.BlockSpec((1, D, FB), lambda e, f, t: (e, 0, f)),
            pl.BlockSpec((1, FB, D), lambda e, f, t: (e, f, 0)),
        ],
        out_specs=pl.BlockSpec((1, BT, D), lambda e, f, t: (e, t, 0)),
        out_shape=jax.ShapeDtypeStruct((E_LOCAL, CAP, D), jnp.bfloat16),
        scratch_shapes=[
            pltpu.VMEM((D, FB), jnp.bfloat16),
            pltpu.VMEM((FB, D), jnp.bfloat16),
            pltpu.VMEM((CAP, D), jnp.float32),
        ],
        compiler_params=pltpu.CompilerParams(vmem_limit_bytes=_VMEM_LIMIT),
    )(xg, w1, w2)


def _scatter_onehot(idx, yg):

    def body(idx_ref, yg_ref, o_ref):
        r = pl.program_id(0)
        e = pl.program_id(1)
        rows = lax.broadcasted_iota(jnp.int32, (CS, CAP), 0) + r * CS
        pt = (rows == idx_ref[e][None, :]).astype(jnp.bfloat16)
        contrib = jnp.dot(pt, yg_ref[0], preferred_element_type=jnp.float32)

        @pl.when(e == 0)
        def _():
            o_ref[...] = contrib.astype(jnp.bfloat16)

        @pl.when(e != 0)
        def _():
            o_ref[...] = o_ref[...] + contrib.astype(jnp.bfloat16)

    return pl.pallas_call(
        body,
        grid=(2 * T // CS, E_LOCAL),
        in_specs=[
            pl.BlockSpec(memory_space=pltpu.VMEM),
            pl.BlockSpec((1, CAP, D), lambda r, e: (e, 0, 0)),
        ],
        out_specs=pl.BlockSpec((CS, D), lambda r, e: (r, 0)),
        out_shape=jax.ShapeDtypeStruct((2 * T, D), jnp.bfloat16),
        compiler_params=pltpu.CompilerParams(vmem_limit_bytes=_VMEM_LIMIT),
    )(idx, yg)


def _exchange_back(theirs_bf):

    def body(theirs_ref, recv_ref, s_sem, r_sem):
        peer = _peer()
        _neighbor_barrier(peer)
        rdma = pltpu.make_async_remote_copy(
            src_ref=theirs_ref, dst_ref=recv_ref, send_sem=s_sem,
            recv_sem=r_sem, device_id=peer, device_id_type=_MESH,
        )
        rdma.start()
        rdma.wait()

    return pl.pallas_call(
        body,
        out_shape=jax.ShapeDtypeStruct((T, D), jnp.bfloat16),
        in_specs=[pl.BlockSpec(memory_space=pltpu.VMEM)],
        out_specs=pl.BlockSpec(memory_space=pltpu.VMEM),
        scratch_shapes=[pltpu.SemaphoreType.DMA, pltpu.SemaphoreType.DMA],
        compiler_params=pltpu.CompilerParams(
            collective_id=1, vmem_limit_bytes=_VMEM_LIMIT
        ),
    )(theirs_bf)


def kernel(x, assign, W1, W2):
    my_x = lax.axis_index("x")
    x_bf = x.astype(jnp.bfloat16)
    a2d = assign.reshape(32, 128)

    x_other, a_other = _exchange(x_bf, a2d)
    assign_all = jnp.concatenate([assign, a_other.reshape(-1)])

    e_base = my_x * E_LOCAL
    idx = jnp.stack(
        [
            jnp.nonzero(assign_all == e_base + i, size=CAP, fill_value=2 * T)[0]
            for i in range(E_LOCAL)
        ]
    ).astype(jnp.int32)

    xg = _gather_onehot(idx, x_bf, x_other)
    yg = _moe_ffn(xg, W1, W2)
    partial = _scatter_onehot(idx, yg)

    recv = _exchange_back(partial[T:])
    return partial[:T].astype(jnp.float32) + recv.astype(jnp.float32)


# device time: 1156561 ns/iter; 2.9637x vs baseline; 1.0282x over previous
import jax
import jax.numpy as jnp
from jax import lax
from jax.experimental import pallas as pl
from jax.experimental.pallas import tpu as pltpu

T = 4096
D = 2048
F = 4096
E_LOCAL = 4
N_EXP = 8
CAP = 1280
BT = 256
FB = 1024
NF = F // FB
CG = 1024
CS = 1024

_MESH = pl.DeviceIdType.MESH
_VMEM_LIMIT = 60 * 1024 * 1024


def _peer():
    return (1 - lax.axis_index("x"), lax.axis_index("y"))


def _neighbor_barrier(peer):
    bar = pltpu.get_barrier_semaphore()
    pl.semaphore_signal(bar, inc=1, device_id=peer, device_id_type=_MESH)
    pl.semaphore_wait(bar, 1)


def _exchange(x_bf, a2d):

    def body(x_ref, a_ref, xo_ref, ao_ref, xs_sem, xr_sem, as_sem, ar_sem):
        peer = _peer()
        _neighbor_barrier(peer)
        rx = pltpu.make_async_remote_copy(
            src_ref=x_ref, dst_ref=xo_ref, send_sem=xs_sem, recv_sem=xr_sem,
            device_id=peer, device_id_type=_MESH,
        )
        ra = pltpu.make_async_remote_copy(
            src_ref=a_ref, dst_ref=ao_ref, send_sem=as_sem, recv_sem=ar_sem,
            device_id=peer, device_id_type=_MESH,
        )
        rx.start()
        ra.start()
        rx.wait()
        ra.wait()

    return pl.pallas_call(
        body,
        out_shape=(
            jax.ShapeDtypeStruct((T, D), jnp.bfloat16),
            jax.ShapeDtypeStruct(a2d.shape, jnp.int32),
        ),
        in_specs=[
            pl.BlockSpec(memory_space=pltpu.VMEM),
            pl.BlockSpec(memory_space=pltpu.VMEM),
        ],
        out_specs=(
            pl.BlockSpec(memory_space=pltpu.VMEM),
            pl.BlockSpec(memory_space=pltpu.VMEM),
        ),
        scratch_shapes=[pltpu.SemaphoreType.DMA] * 4,
        compiler_params=pltpu.CompilerParams(
            collective_id=0, vmem_limit_bytes=_VMEM_LIMIT
        ),
    )(x_bf, a2d)


def _gather_onehot(s_mine, s_other, x_mine, x_other):

    def body(sm_ref, so_ref, xm_ref, xo_ref, o_ref):
        e = pl.program_id(0)
        c = pl.program_id(1)
        slot = e * CAP + lax.broadcasted_iota(jnp.int32, (CAP, CG), 0)
        pm = (slot == sm_ref[c][None, :]).astype(jnp.bfloat16)
        po = (slot == so_ref[c][None, :]).astype(jnp.bfloat16)
        contrib = jnp.dot(
            pm, xm_ref[...], preferred_element_type=jnp.float32
        ) + jnp.dot(po, xo_ref[...], preferred_element_type=jnp.float32)

        @pl.when(c == 0)
        def _():
            o_ref[0] = contrib.astype(jnp.bfloat16)

        @pl.when(c != 0)
        def _():
            o_ref[0] = o_ref[0] + contrib.astype(jnp.bfloat16)

    return pl.pallas_call(
        body,
        grid=(E_LOCAL, T // CG),
        in_specs=[
            pl.BlockSpec(memory_space=pltpu.VMEM),
            pl.BlockSpec(memory_space=pltpu.VMEM),
            pl.BlockSpec((CG, D), lambda e, c: (c, 0)),
            pl.BlockSpec((CG, D), lambda e, c: (c, 0)),
        ],
        out_specs=pl.BlockSpec((1, CAP, D), lambda e, c: (e, 0, 0)),
        out_shape=jax.ShapeDtypeStruct((E_LOCAL, CAP, D), jnp.bfloat16),
        compiler_params=pltpu.CompilerParams(vmem_limit_bytes=_VMEM_LIMIT),
    )(s_mine, s_other, x_mine, x_other)


def _moe_ffn(xg, w1, w2):

    def body(x_ref, w1_ref, w2_ref, o_ref, w1b, w2b, acc_ref):
        f = pl.program_id(1)
        t = pl.program_id(2)

        @pl.when(t == 0)
        def _():
            w1b[...] = w1_ref[0].astype(jnp.bfloat16)
            w2b[...] = w2_ref[0].astype(jnp.bfloat16)

        h = jnp.dot(x_ref[0], w1b[...], preferred_element_type=jnp.float32)
        h = jnp.maximum(h, 0.0).astype(jnp.bfloat16)
        p = jnp.dot(h, w2b[...], preferred_element_type=jnp.float32)
        sl = pl.ds(t * BT, BT)

        @pl.when(f == 0)
        def _():
            acc_ref[sl, :] = p

        @pl.when(f != 0)
        def _():
            acc_ref[sl, :] += p

        @pl.when(f == NF - 1)
        def _():
            o_ref[0] = acc_ref[sl, :].astype(jnp.bfloat16)

    return pl.pallas_call(
        body,
        grid=(E_LOCAL, NF, CAP // BT),
        in_specs=[
            pl.BlockSpec((1, BT, D), lambda e, f, t: (e, t, 0)),
            pl.BlockSpec((1, D, FB), lambda e, f, t: (e, 0, f)),
            pl.BlockSpec((1, FB, D), lambda e, f, t: (e, f, 0)),
        ],
        out_specs=pl.BlockSpec((1, BT, D), lambda e, f, t: (e, t, 0)),
        out_shape=jax.ShapeDtypeStruct((E_LOCAL, CAP, D), jnp.bfloat16),
        scratch_shapes=[
            pltpu.VMEM((D, FB), jnp.bfloat16),
            pltpu.VMEM((FB, D), jnp.bfloat16),
            pltpu.VMEM((CAP, D), jnp.float32),
        ],
        compiler_params=pltpu.CompilerParams(vmem_limit_bytes=_VMEM_LIMIT),
    )(xg, w1, w2)


def _scatter_onehot(s_all, yg):

    def body(s_ref, yg_ref, o_ref):
        r = pl.program_id(0)
        e = pl.program_id(1)
        slot = e * CAP + lax.broadcasted_iota(jnp.int32, (CS, CAP), 1)
        pt = (s_ref[r][:, None] == slot).astype(jnp.bfloat16)
        contrib = jnp.dot(pt, yg_ref[0], preferred_element_type=jnp.float32)

        @pl.when(e == 0)
        def _():
            o_ref[...] = contrib.astype(jnp.bfloat16)

        @pl.when(e != 0)
        def _():
            o_ref[...] = o_ref[...] + contrib.astype(jnp.bfloat16)

    return pl.pallas_call(
        body,
        grid=(2 * T // CS, E_LOCAL),
        in_specs=[
            pl.BlockSpec(memory_space=pltpu.VMEM),
            pl.BlockSpec((1, CAP, D), lambda r, e: (e, 0, 0)),
        ],
        out_specs=pl.BlockSpec((CS, D), lambda r, e: (r, 0)),
        out_shape=jax.ShapeDtypeStruct((2 * T, D), jnp.bfloat16),
        compiler_params=pltpu.CompilerParams(vmem_limit_bytes=_VMEM_LIMIT),
    )(s_all, yg)


def _exchange_back(theirs_bf):

    def body(theirs_ref, recv_ref, s_sem, r_sem):
        peer = _peer()
        _neighbor_barrier(peer)
        rdma = pltpu.make_async_remote_copy(
            src_ref=theirs_ref, dst_ref=recv_ref, send_sem=s_sem,
            recv_sem=r_sem, device_id=peer, device_id_type=_MESH,
        )
        rdma.start()
        rdma.wait()

    return pl.pallas_call(
        body,
        out_shape=jax.ShapeDtypeStruct((T, D), jnp.bfloat16),
        in_specs=[pl.BlockSpec(memory_space=pltpu.VMEM)],
        out_specs=pl.BlockSpec(memory_space=pltpu.VMEM),
        scratch_shapes=[pltpu.SemaphoreType.DMA, pltpu.SemaphoreType.DMA],
        compiler_params=pltpu.CompilerParams(
            collective_id=1, vmem_limit_bytes=_VMEM_LIMIT
        ),
    )(theirs_bf)


def _slot_ids(assign_all, e_base):
    oh = (
        assign_all[None, :] == jnp.arange(N_EXP, dtype=jnp.int32)[:, None]
    ).astype(jnp.int32)
    rank = (oh * jnp.cumsum(oh, axis=1)).sum(0) - 1
    loc = assign_all - e_base
    valid = (loc >= 0) & (loc < E_LOCAL) & (rank < CAP)
    return jnp.where(valid, loc * CAP + rank, -1).astype(jnp.int32)


def kernel(x, assign, W1, W2):
    my_x = lax.axis_index("x")
    x_bf = x.astype(jnp.bfloat16)
    a2d = assign.reshape(32, 128)

    x_other, a_other = _exchange(x_bf, a2d)
    assign_all = jnp.concatenate([assign, a_other.reshape(-1)])

    s_all = _slot_ids(assign_all, my_x * E_LOCAL)
    s2d = s_all.reshape(2 * T // CG, CG)

    xg = _gather_onehot(s2d[: T // CG], s2d[T // CG:], x_bf, x_other)
    yg = _moe_ffn(xg, W1, W2)
    partial = _scatter_onehot(s_all.reshape(2 * T // CS, CS), yg)

    recv = _exchange_back(partial[T:])
    return partial[:T].astype(jnp.float32) + recv.astype(jnp.float32)


# device time: 1102611 ns/iter; 3.1087x vs baseline; 1.0489x over previous
import jax
import jax.numpy as jnp
from jax import lax
from jax.experimental import pallas as pl
from jax.experimental.pallas import tpu as pltpu

T = 4096
D = 2048
F = 4096
E_LOCAL = 4
N_EXP = 8
CAP = 1280
CAPH = CAP // 2
BT = 320
FB = 1024
NF = F // FB
CG = 1024
CS = 1024

_MESH = pl.DeviceIdType.MESH
_VMEM_LIMIT = 60 * 1024 * 1024


def _x_peer():
    return (1 - lax.axis_index("x"), lax.axis_index("y"))


def _y_peer():
    return (lax.axis_index("x"), 1 - lax.axis_index("y"))


def _neighbor_barrier(peer):
    bar = pltpu.get_barrier_semaphore()
    pl.semaphore_signal(bar, inc=1, device_id=peer, device_id_type=_MESH)
    pl.semaphore_wait(bar, 1)


def _exchange(x_bf, a2d):

    def body(x_ref, a_ref, xo_ref, ao_ref, xs_sem, xr_sem, as_sem, ar_sem):
        peer = _x_peer()
        _neighbor_barrier(peer)
        rx = pltpu.make_async_remote_copy(
            src_ref=x_ref, dst_ref=xo_ref, send_sem=xs_sem, recv_sem=xr_sem,
            device_id=peer, device_id_type=_MESH,
        )
        ra = pltpu.make_async_remote_copy(
            src_ref=a_ref, dst_ref=ao_ref, send_sem=as_sem, recv_sem=ar_sem,
            device_id=peer, device_id_type=_MESH,
        )
        rx.start()
        ra.start()
        rx.wait()
        ra.wait()

    return pl.pallas_call(
        body,
        out_shape=(
            jax.ShapeDtypeStruct((T, D), jnp.bfloat16),
            jax.ShapeDtypeStruct(a2d.shape, jnp.int32),
        ),
        in_specs=[
            pl.BlockSpec(memory_space=pltpu.VMEM),
            pl.BlockSpec(memory_space=pltpu.VMEM),
        ],
        out_specs=(
            pl.BlockSpec(memory_space=pltpu.VMEM),
            pl.BlockSpec(memory_space=pltpu.VMEM),
        ),
        scratch_shapes=[pltpu.SemaphoreType.DMA] * 4,
        compiler_params=pltpu.CompilerParams(
            collective_id=0, vmem_limit_bytes=_VMEM_LIMIT
        ),
    )(x_bf, a2d)


def _gather_onehot(s_mine, s_other, x_mine, x_other):

    def body(sm_ref, so_ref, xm_ref, xo_ref, o_ref):
        e = pl.program_id(0)
        c = pl.program_id(1)
        my_y = lax.axis_index("y")
        slot = (
            e * CAP + my_y * CAPH
            + lax.broadcasted_iota(jnp.int32, (CAPH, CG), 0)
        )
        pm = (slot == sm_ref[c][None, :]).astype(jnp.bfloat16)
        po = (slot == so_ref[c][None, :]).astype(jnp.bfloat16)
        contrib = jnp.dot(
            pm, xm_ref[...], preferred_element_type=jnp.float32
        ) + jnp.dot(po, xo_ref[...], preferred_element_type=jnp.float32)

        @pl.when(c == 0)
        def _():
            o_ref[0] = contrib.astype(jnp.bfloat16)

        @pl.when(c != 0)
        def _():
            o_ref[0] = o_ref[0] + contrib.astype(jnp.bfloat16)

    return pl.pallas_call(
        body,
        grid=(E_LOCAL, T // CG),
        in_specs=[
            pl.BlockSpec(memory_space=pltpu.VMEM),
            pl.BlockSpec(memory_space=pltpu.VMEM),
            pl.BlockSpec((CG, D), lambda e, c: (c, 0)),
            pl.BlockSpec((CG, D), lambda e, c: (c, 0)),
        ],
        out_specs=pl.BlockSpec((1, CAPH, D), lambda e, c: (e, 0, 0)),
        out_shape=jax.ShapeDtypeStruct((E_LOCAL, CAPH, D), jnp.bfloat16),
        compiler_params=pltpu.CompilerParams(vmem_limit_bytes=_VMEM_LIMIT),
    )(s_mine, s_other, x_mine, x_other)


def _moe_ffn(xg, w1, w2):

    def body(x_ref, w1_ref, w2_ref, o_ref, w1b, w2b, acc_ref):
        f = pl.program_id(1)
        t = pl.program_id(2)

        @pl.when(t == 0)
        def _():
            w1b[...] = w1_ref[0].astype(jnp.bfloat16)
            w2b[...] = w2_ref[0].astype(jnp.bfloat16)

        h = jnp.dot(x_ref[0], w1b[...], preferred_element_type=jnp.float32)
        h = jnp.maximum(h, 0.0).astype(jnp.bfloat16)
        p = jnp.dot(h, w2b[...], preferred_element_type=jnp.float32)
        sl = pl.ds(t * BT, BT)

        @pl.when(f == 0)
        def _():
            acc_ref[sl, :] = p

        @pl.when(f != 0)
        def _():
            acc_ref[sl, :] += p

        @pl.when(f == NF - 1)
        def _():
            o_ref[0] = acc_ref[sl, :].astype(jnp.bfloat16)

    return pl.pallas_call(
        body,
        grid=(E_LOCAL, NF, CAPH // BT),
        in_specs=[
            pl.BlockSpec((1, BT, D), lambda e, f, t: (e, t, 0)),
            pl.BlockSpec((1, D, FB), lambda e, f, t: (e, 0, f)),
            pl.BlockSpec((1, FB, D), lambda e, f, t: (e, f, 0)),
        ],
        out_specs=pl.BlockSpec((1, BT, D), lambda e, f, t: (e, t, 0)),
        out_shape=jax.ShapeDtypeStruct((E_LOCAL, CAPH, D), jnp.bfloat16),
        scratch_shapes=[
            pltpu.VMEM((D, FB), jnp.bfloat16),
            pltpu.VMEM((FB, D), jnp.bfloat16),
            pltpu.VMEM((CAPH, D), jnp.float32),
        ],
        compiler_params=pltpu.CompilerParams(vmem_limit_bytes=_VMEM_LIMIT),
    )(xg, w1, w2)


def _exchange_y(ygh):

    def body(ygh_ref, recv_ref, s_sem, r_sem):
        peer = _y_peer()
        _neighbor_barrier(peer)
        rdma = pltpu.make_async_remote_copy(
            src_ref=ygh_ref, dst_ref=recv_ref, send_sem=s_sem,
            recv_sem=r_sem, device_id=peer, device_id_type=_MESH,
        )
        rdma.start()
        rdma.wait()

    return pl.pallas_call(
        body,
        out_shape=jax.ShapeDtypeStruct((E_LOCAL, CAPH, D), jnp.bfloat16),
        in_specs=[pl.BlockSpec(memory_space=pltpu.VMEM)],
        out_specs=pl.BlockSpec(memory_space=pltpu.VMEM),
        scratch_shapes=[pltpu.SemaphoreType.DMA, pltpu.SemaphoreType.DMA],
        compiler_params=pltpu.CompilerParams(
            collective_id=2, vmem_limit_bytes=_VMEM_LIMIT
        ),
    )(ygh)


def _scatter_onehot(s_all, ygh_own, ygh_recv):

    def body(s_ref, yo_ref, yr_ref, o_ref):
        r = pl.program_id(0)
        e = pl.program_id(1)
        my_y = lax.axis_index("y")
        s_col = s_ref[r][:, None]
        rk = lax.broadcasted_iota(jnp.int32, (CS, CAPH), 1)
        p_own = (s_col == e * CAP + my_y * CAPH + rk).astype(jnp.bfloat16)
        p_recv = (s_col == e * CAP + (1 - my_y) * CAPH + rk).astype(
            jnp.bfloat16
        )
        contrib = jnp.dot(
            p_own, yo_ref[0], preferred_element_type=jnp.float32
        ) + jnp.dot(p_recv, yr_ref[0], preferred_element_type=jnp.float32)

        @pl.when(e == 0)
        def _():
            o_ref[...] = contrib.astype(jnp.bfloat16)

        @pl.when(e != 0)
        def _():
            o_ref[...] = o_ref[...] + contrib.astype(jnp.bfloat16)

    return pl.pallas_call(
        body,
        grid=(2 * T // CS, E_LOCAL),
        in_specs=[
            pl.BlockSpec(memory_space=pltpu.VMEM),
            pl.BlockSpec((1, CAPH, D), lambda r, e: (e, 0, 0)),
            pl.BlockSpec((1, CAPH, D), lambda r, e: (e, 0, 0)),
        ],
        out_specs=pl.BlockSpec((CS, D), lambda r, e: (r, 0)),
        out_shape=jax.ShapeDtypeStruct((2 * T, D), jnp.bfloat16),
        compiler_params=pltpu.CompilerParams(vmem_limit_bytes=_VMEM_LIMIT),
    )(s_all, ygh_own, ygh_recv)


def _exchange_back(theirs_bf):

    def body(theirs_ref, recv_ref, s_sem, r_sem):
        peer = _x_peer()
        _neighbor_barrier(peer)
        rdma = pltpu.make_async_remote_copy(
            src_ref=theirs_ref, dst_ref=recv_ref, send_sem=s_sem,
            recv_sem=r_sem, device_id=peer, device_id_type=_MESH,
        )
        rdma.start()
        rdma.wait()

    return pl.pallas_call(
        body,
        out_shape=jax.ShapeDtypeStruct((T, D), jnp.bfloat16),
        in_specs=[pl.BlockSpec(memory_space=pltpu.VMEM)],
        out_specs=pl.BlockSpec(memory_space=pltpu.VMEM),
        scratch_shapes=[pltpu.SemaphoreType.DMA, pltpu.SemaphoreType.DMA],
        compiler_params=pltpu.CompilerParams(
            collective_id=1, vmem_limit_bytes=_VMEM_LIMIT
        ),
    )(theirs_bf)


def _slot_ids(assign_all, e_base):
    oh = (
        assign_all[None, :] == jnp.arange(N_EXP, dtype=jnp.int32)[:, None]
    ).astype(jnp.int32)
    rank = (oh * jnp.cumsum(oh, axis=1)).sum(0) - 1
    loc = assign_all - e_base
    valid = (loc >= 0) & (loc < E_LOCAL) & (rank < CAP)
    return jnp.where(valid, loc * CAP + rank, -1).astype(jnp.int32)


def kernel(x, assign, W1, W2):
    my_x = lax.axis_index("x")
    x_bf = x.astype(jnp.bfloat16)
    a2d = assign.reshape(32, 128)

    x_other, a_other = _exchange(x_bf, a2d)
    assign_all = jnp.concatenate([assign, a_other.reshape(-1)])

    s_all = _slot_ids(assign_all, my_x * E_LOCAL)
    s2d = s_all.reshape(2 * T // CG, CG)

    xg = _gather_onehot(s2d[: T // CG], s2d[T // CG:], x_bf, x_other)
    ygh = _moe_ffn(xg, W1, W2)
    ygh_recv = _exchange_y(ygh)
    partial = _scatter_onehot(s_all.reshape(2 * T // CS, CS), ygh, ygh_recv)

    recv = _exchange_back(partial[T:])
    return partial[:T].astype(jnp.float32) + recv.astype(jnp.float32)


# device time: 971070 ns/iter; 3.5298x vs baseline; 1.1355x over previous
import jax
import jax.numpy as jnp
from jax import lax
from jax.experimental import pallas as pl
from jax.experimental.pallas import tpu as pltpu

T = 4096
D = 2048
F = 4096
E_LOCAL = 4
N_EXP = 8
CAP = 1280
CAPH = CAP // 2
BT = 320
FB = 1024
NF = F // FB
CG = 1024
CS = 1024

_MESH = pl.DeviceIdType.MESH
_VMEM_LIMIT = 60 * 1024 * 1024


def _x_peer():
    return (1 - lax.axis_index("x"), lax.axis_index("y"))


def _y_peer():
    return (lax.axis_index("x"), 1 - lax.axis_index("y"))


def _neighbor_barrier(peer):
    bar = pltpu.get_barrier_semaphore()
    pl.semaphore_signal(bar, inc=1, device_id=peer, device_id_type=_MESH)
    pl.semaphore_wait(bar, 1)


def _exchange(x_bf, a2d):

    def body(x_ref, a_ref, xo_ref, ao_ref, xs_sem, xr_sem, as_sem, ar_sem):
        peer = _x_peer()
        _neighbor_barrier(peer)
        rx = pltpu.make_async_remote_copy(
            src_ref=x_ref, dst_ref=xo_ref, send_sem=xs_sem, recv_sem=xr_sem,
            device_id=peer, device_id_type=_MESH,
        )
        ra = pltpu.make_async_remote_copy(
            src_ref=a_ref, dst_ref=ao_ref, send_sem=as_sem, recv_sem=ar_sem,
            device_id=peer, device_id_type=_MESH,
        )
        rx.start()
        ra.start()
        rx.wait()
        ra.wait()

    return pl.pallas_call(
        body,
        out_shape=(
            jax.ShapeDtypeStruct((T, D), jnp.bfloat16),
            jax.ShapeDtypeStruct(a2d.shape, jnp.int32),
        ),
        in_specs=[
            pl.BlockSpec(memory_space=pltpu.VMEM),
            pl.BlockSpec(memory_space=pltpu.VMEM),
        ],
        out_specs=(
            pl.BlockSpec(memory_space=pltpu.VMEM),
            pl.BlockSpec(memory_space=pltpu.VMEM),
        ),
        scratch_shapes=[pltpu.SemaphoreType.DMA] * 4,
        compiler_params=pltpu.CompilerParams(
            collective_id=0, vmem_limit_bytes=_VMEM_LIMIT
        ),
    )(x_bf, a2d)


def _gather_onehot(s_mine, s_other, x_mine, x_other):

    def body(sm_ref, so_ref, xm_ref, xo_ref, o_ref):
        e = pl.program_id(0)
        c = pl.program_id(1)
        my_y = lax.axis_index("y")
        slot = (
            e * CAP + my_y * CAPH
            + lax.broadcasted_iota(jnp.int32, (CAPH, CG), 0)
        )
        pm = (slot == sm_ref[c][None, :]).astype(jnp.bfloat16)
        po = (slot == so_ref[c][None, :]).astype(jnp.bfloat16)
        contrib = jnp.dot(
            pm, xm_ref[...], preferred_element_type=jnp.float32
        ) + jnp.dot(po, xo_ref[...], preferred_element_type=jnp.float32)

        @pl.when(c == 0)
        def _():
            o_ref[0] = contrib.astype(jnp.bfloat16)

        @pl.when(c != 0)
        def _():
            o_ref[0] = o_ref[0] + contrib.astype(jnp.bfloat16)

    return pl.pallas_call(
        body,
        grid=(E_LOCAL, T // CG),
        in_specs=[
            pl.BlockSpec(memory_space=pltpu.VMEM),
            pl.BlockSpec(memory_space=pltpu.VMEM),
            pl.BlockSpec((CG, D), lambda e, c: (c, 0)),
            pl.BlockSpec((CG, D), lambda e, c: (c, 0)),
        ],
        out_specs=pl.BlockSpec((1, CAPH, D), lambda e, c: (e, 0, 0)),
        out_shape=jax.ShapeDtypeStruct((E_LOCAL, CAPH, D), jnp.bfloat16),
        compiler_params=pltpu.CompilerParams(vmem_limit_bytes=_VMEM_LIMIT),
    )(s_mine, s_other, x_mine, x_other)


def _moe_ffn(xg, w1, w2):

    def body(x_ref, w1_ref, w2_ref, o_ref, w1b, w2b, acc_ref):
        f = pl.program_id(1)
        t = pl.program_id(2)

        @pl.when(t == 0)
        def _():
            w1b[...] = w1_ref[0].astype(jnp.bfloat16)
            w2b[...] = w2_ref[0].astype(jnp.bfloat16)

        h = jnp.dot(x_ref[0], w1b[...], preferred_element_type=jnp.float32)
        h = jnp.maximum(h, 0.0).astype(jnp.bfloat16)
        p = jnp.dot(h, w2b[...], preferred_element_type=jnp.float32)
        sl = pl.ds(t * BT, BT)

        @pl.when(f == 0)
        def _():
            acc_ref[sl, :] = p

        @pl.when(f != 0)
        def _():
            acc_ref[sl, :] += p

        @pl.when(f == NF - 1)
        def _():
            o_ref[0] = acc_ref[sl, :].astype(jnp.bfloat16)

    return pl.pallas_call(
        body,
        grid=(E_LOCAL, NF, CAPH // BT),
        in_specs=[
            pl.BlockSpec((1, BT, D), lambda e, f, t: (e, t, 0)),
            pl.BlockSpec((1, D, FB), lambda e, f, t: (e, 0, f)),
            pl.BlockSpec((1, FB, D), lambda e, f, t: (e, f, 0)),
        ],
        out_specs=pl.BlockSpec((1, BT, D), lambda e, f, t: (e, t, 0)),
        out_shape=jax.ShapeDtypeStruct((E_LOCAL, CAPH, D), jnp.bfloat16),
        scratch_shapes=[
            pltpu.VMEM((D, FB), jnp.bfloat16),
            pltpu.VMEM((FB, D), jnp.bfloat16),
            pltpu.VMEM((CAPH, D), jnp.float32),
        ],
        compiler_params=pltpu.CompilerParams(vmem_limit_bytes=_VMEM_LIMIT),
    )(xg, w1, w2)


def _exchange_y(ygh):

    def body(ygh_ref, recv_ref, s_sem, r_sem):
        peer = _y_peer()
        _neighbor_barrier(peer)
        rdma = pltpu.make_async_remote_copy(
            src_ref=ygh_ref, dst_ref=recv_ref, send_sem=s_sem,
            recv_sem=r_sem, device_id=peer, device_id_type=_MESH,
        )
        rdma.start()
        rdma.wait()

    return pl.pallas_call(
        body,
        out_shape=jax.ShapeDtypeStruct((E_LOCAL, CAPH, D), jnp.bfloat16),
        in_specs=[pl.BlockSpec(memory_space=pltpu.VMEM)],
        out_specs=pl.BlockSpec(memory_space=pltpu.VMEM),
        scratch_shapes=[pltpu.SemaphoreType.DMA, pltpu.SemaphoreType.DMA],
        compiler_params=pltpu.CompilerParams(
            collective_id=2, vmem_limit_bytes=_VMEM_LIMIT
        ),
    )(ygh)


def _scatter_contrib(s_ref, chunk, e, yo_ref, yr_ref, my_y):
    s_col = s_ref[chunk][:, None]
    rk = lax.broadcasted_iota(jnp.int32, (CS, CAPH), 1)
    p_own = (s_col == e * CAP + my_y * CAPH + rk).astype(jnp.bfloat16)
    p_recv = (s_col == e * CAP + (1 - my_y) * CAPH + rk).astype(jnp.bfloat16)
    return jnp.dot(
        p_own, yo_ref[0], preferred_element_type=jnp.float32
    ) + jnp.dot(p_recv, yr_ref[0], preferred_element_type=jnp.float32)


def _scatter_theirs(s_all, ygh_own, ygh_recv):

    def body(s_ref, yo_ref, yr_ref, o_ref):
        r = pl.program_id(0)
        e = pl.program_id(1)
        my_y = lax.axis_index("y")
        contrib = _scatter_contrib(s_ref, r + T // CS, e, yo_ref, yr_ref, my_y)

        @pl.when(e == 0)
        def _():
            o_ref[...] = contrib.astype(jnp.bfloat16)

        @pl.when(e != 0)
        def _():
            o_ref[...] = o_ref[...] + contrib.astype(jnp.bfloat16)

    return pl.pallas_call(
        body,
        grid=(T // CS, E_LOCAL),
        in_specs=[
            pl.BlockSpec(memory_space=pltpu.VMEM),
            pl.BlockSpec((1, CAPH, D), lambda r, e: (e, 0, 0)),
            pl.BlockSpec((1, CAPH, D), lambda r, e: (e, 0, 0)),
        ],
        out_specs=pl.BlockSpec((CS, D), lambda r, e: (r, 0)),
        out_shape=jax.ShapeDtypeStruct((T, D), jnp.bfloat16),
        compiler_params=pltpu.CompilerParams(vmem_limit_bytes=_VMEM_LIMIT),
    )(s_all, ygh_own, ygh_recv)


def _scatter_mine_exchange(theirs_bf, s_all, ygh_own, ygh_recv):
    n_r = T // CS

    def body(pt_ref, s_ref, yo_ref, yr_ref, om_ref, xrecv_ref, s_sem, r_sem):
        r = pl.program_id(0)
        e = pl.program_id(1)
        my_y = lax.axis_index("y")
        peer = _x_peer()
        rdma = pltpu.make_async_remote_copy(
            src_ref=pt_ref, dst_ref=xrecv_ref, send_sem=s_sem,
            recv_sem=r_sem, device_id=peer, device_id_type=_MESH,
        )

        @pl.when((r == 0) & (e == 0))
        def _():
            _neighbor_barrier(peer)
            rdma.start()

        contrib = _scatter_contrib(s_ref, r, e, yo_ref, yr_ref, my_y)

        @pl.when(e == 0)
        def _():
            om_ref[...] = contrib.astype(jnp.bfloat16)

        @pl.when(e != 0)
        def _():
            om_ref[...] = om_ref[...] + contrib.astype(jnp.bfloat16)

        @pl.when((r == n_r - 1) & (e == E_LOCAL - 1))
        def _():
            rdma.wait()

    return pl.pallas_call(
        body,
        grid=(n_r, E_LOCAL),
        in_specs=[
            pl.BlockSpec(memory_space=pltpu.VMEM),
            pl.BlockSpec(memory_space=pltpu.VMEM),
            pl.BlockSpec((1, CAPH, D), lambda r, e: (e, 0, 0)),
            pl.BlockSpec((1, CAPH, D), lambda r, e: (e, 0, 0)),
        ],
        out_specs=(
            pl.BlockSpec((CS, D), lambda r, e: (r, 0)),
            pl.BlockSpec(memory_space=pl.ANY),
        ),
        out_shape=(
            jax.ShapeDtypeStruct((T, D), jnp.bfloat16),
            jax.ShapeDtypeStruct((T, D), jnp.bfloat16),
        ),
        scratch_shapes=[pltpu.SemaphoreType.DMA, pltpu.SemaphoreType.DMA],
        compiler_params=pltpu.CompilerParams(
            collective_id=1, vmem_limit_bytes=_VMEM_LIMIT
        ),
    )(theirs_bf, s_all, ygh_own, ygh_recv)


def _slot_ids(assign_all, e_base):
    oh = (
        assign_all[None, :] == jnp.arange(N_EXP, dtype=jnp.int32)[:, None]
    ).astype(jnp.int32)
    rank = (oh * jnp.cumsum(oh, axis=1)).sum(0) - 1
    loc = assign_all - e_base
    valid = (loc >= 0) & (loc < E_LOCAL) & (rank < CAP)
    return jnp.where(valid, loc * CAP + rank, -1).astype(jnp.int32)


def kernel(x, assign, W1, W2):
    my_x = lax.axis_index("x")
    x_bf = x.astype(jnp.bfloat16)
    a2d = assign.reshape(32, 128)

    x_other, a_other = _exchange(x_bf, a2d)
    assign_all = jnp.concatenate([assign, a_other.reshape(-1)])

    s_all = _slot_ids(assign_all, my_x * E_LOCAL)
    s2d = s_all.reshape(2 * T // CG, CG)

    xg = _gather_onehot(s2d[: T // CG], s2d[T // CG:], x_bf, x_other)
    ygh = _moe_ffn(xg, W1, W2)
    ygh_recv = _exchange_y(ygh)

    s_cs = s_all.reshape(2 * T // CS, CS)
    theirs = _scatter_theirs(s_cs, ygh, ygh_recv)
    mine, recv = _scatter_mine_exchange(theirs, s_cs, ygh, ygh_recv)
    return mine.astype(jnp.float32) + recv.astype(jnp.float32)


# device time: 919495 ns/iter; 3.7278x vs baseline; 1.0561x over previous
import jax
import jax.numpy as jnp
from jax import lax
from jax.experimental import pallas as pl
from jax.experimental.pallas import tpu as pltpu

T = 4096
D = 2048
F = 4096
E_LOCAL = 4
N_EXP = 8
CAP = 1280
CAPH = CAP // 2
BT = 320
FB = 1024
NF = F // FB
CG = 1024
CS = 1024
CSY = 512

_MESH = pl.DeviceIdType.MESH
_VMEM_LIMIT = 60 * 1024 * 1024


def _x_peer():
    return (1 - lax.axis_index("x"), lax.axis_index("y"))


def _y_peer():
    return (lax.axis_index("x"), 1 - lax.axis_index("y"))


def _neighbor_barrier(peer):
    bar = pltpu.get_barrier_semaphore()
    pl.semaphore_signal(bar, inc=1, device_id=peer, device_id_type=_MESH)
    pl.semaphore_wait(bar, 1)


def _exchange(x_bf, a2d):

    def body(x_ref, a_ref, xo_ref, ao_ref, xs_sem, xr_sem, as_sem, ar_sem):
        peer = _x_peer()
        _neighbor_barrier(peer)
        rx = pltpu.make_async_remote_copy(
            src_ref=x_ref, dst_ref=xo_ref, send_sem=xs_sem, recv_sem=xr_sem,
            device_id=peer, device_id_type=_MESH,
        )
        ra = pltpu.make_async_remote_copy(
            src_ref=a_ref, dst_ref=ao_ref, send_sem=as_sem, recv_sem=ar_sem,
            device_id=peer, device_id_type=_MESH,
        )
        rx.start()
        ra.start()
        rx.wait()
        ra.wait()

    return pl.pallas_call(
        body,
        out_shape=(
            jax.ShapeDtypeStruct((T, D), jnp.bfloat16),
            jax.ShapeDtypeStruct(a2d.shape, jnp.int32),
        ),
        in_specs=[
            pl.BlockSpec(memory_space=pltpu.VMEM),
            pl.BlockSpec(memory_space=pltpu.VMEM),
        ],
        out_specs=(
            pl.BlockSpec(memory_space=pltpu.VMEM),
            pl.BlockSpec(memory_space=pltpu.VMEM),
        ),
        scratch_shapes=[pltpu.SemaphoreType.DMA] * 4,
        compiler_params=pltpu.CompilerParams(
            collective_id=0, vmem_limit_bytes=_VMEM_LIMIT
        ),
    )(x_bf, a2d)


def _gather_onehot(s_mine, s_other, x_mine, x_other):

    def body(sm_ref, so_ref, xm_ref, xo_ref, o_ref):
        e = pl.program_id(0)
        c = pl.program_id(1)
        my_y = lax.axis_index("y")
        slot = (
            e * CAP + my_y * CAPH
            + lax.broadcasted_iota(jnp.int32, (CAPH, CG), 0)
        )
        pm = (slot == sm_ref[c][None, :]).astype(jnp.bfloat16)
        po = (slot == so_ref[c][None, :]).astype(jnp.bfloat16)
        contrib = jnp.dot(
            pm, xm_ref[...], preferred_element_type=jnp.float32
        ) + jnp.dot(po, xo_ref[...], preferred_element_type=jnp.float32)

        @pl.when(c == 0)
        def _():
            o_ref[0] = contrib.astype(jnp.bfloat16)

        @pl.when(c != 0)
        def _():
            o_ref[0] = o_ref[0] + contrib.astype(jnp.bfloat16)

    return pl.pallas_call(
        body,
        grid=(E_LOCAL, T // CG),
        in_specs=[
            pl.BlockSpec(memory_space=pltpu.VMEM),
            pl.BlockSpec(memory_space=pltpu.VMEM),
            pl.BlockSpec((CG, D), lambda e, c: (c, 0)),
            pl.BlockSpec((CG, D), lambda e, c: (c, 0)),
        ],
        out_specs=pl.BlockSpec((1, CAPH, D), lambda e, c: (e, 0, 0)),
        out_shape=jax.ShapeDtypeStruct((E_LOCAL, CAPH, D), jnp.bfloat16),
        compiler_params=pltpu.CompilerParams(vmem_limit_bytes=_VMEM_LIMIT),
    )(s_mine, s_other, x_mine, x_other)


def _moe_ffn(xg, w1, w2):

    def body(x_ref, w1_ref, w2_ref, o_ref, w1b, w2b, acc_ref):
        f = pl.program_id(1)
        t = pl.program_id(2)

        @pl.when(t == 0)
        def _():
            w1b[...] = w1_ref[0].astype(jnp.bfloat16)
            w2b[...] = w2_ref[0].astype(jnp.bfloat16)

        h = jnp.dot(x_ref[0], w1b[...], preferred_element_type=jnp.float32)
        h = jnp.maximum(h, 0.0).astype(jnp.bfloat16)
        p = jnp.dot(h, w2b[...], preferred_element_type=jnp.float32)
        sl = pl.ds(t * BT, BT)

        @pl.when(f == 0)
        def _():
            acc_ref[sl, :] = p

        @pl.when(f != 0)
        def _():
            acc_ref[sl, :] += p

        @pl.when(f == NF - 1)
        def _():
            o_ref[0] = acc_ref[sl, :].astype(jnp.bfloat16)

    return pl.pallas_call(
        body,
        grid=(E_LOCAL, NF, CAPH // BT),
        in_specs=[
            pl.BlockSpec((1, BT, D), lambda e, f, t: (e, t, 0)),
            pl.BlockSpec((1, D, FB), lambda e, f, t: (e, 0, f)),
            pl.BlockSpec((1, FB, D), lambda e, f, t: (e, f, 0)),
        ],
        out_specs=pl.BlockSpec((1, BT, D), lambda e, f, t: (e, t, 0)),
        out_shape=jax.ShapeDtypeStruct((E_LOCAL, CAPH, D), jnp.bfloat16),
        scratch_shapes=[
            pltpu.VMEM((D, FB), jnp.bfloat16),
            pltpu.VMEM((FB, D), jnp.bfloat16),
            pltpu.VMEM((CAPH, D), jnp.float32),
        ],
        compiler_params=pltpu.CompilerParams(vmem_limit_bytes=_VMEM_LIMIT),
    )(xg, w1, w2)


def _onehot_rows(s_ref, chunk, base, n_slots):
    s_col = s_ref[chunk][:, None]
    rows = s_ref.shape[1]
    rk = lax.broadcasted_iota(jnp.int32, (rows, n_slots), 1)
    return (s_col == base + rk).astype(jnp.bfloat16)


def _scatter_contrib(s_ref, chunk, e, yo_ref, yr_ref, my_y):
    p_own = _onehot_rows(s_ref, chunk, e * CAP + my_y * CAPH, CAPH)
    p_recv = _onehot_rows(s_ref, chunk, e * CAP + (1 - my_y) * CAPH, CAPH)
    return jnp.dot(
        p_own, yo_ref[0], preferred_element_type=jnp.float32
    ) + jnp.dot(p_recv, yr_ref[0], preferred_element_type=jnp.float32)


def _scatter_theirs_yexchange(ygh, s_all):
    n_r = T // CSY

    def body(ygw_ref, s_ref, yo_ref, o_ref, yrecv_ref, acc_ref, s_sem, r_sem):
        p = pl.program_id(0)
        r = pl.program_id(1)
        e = pl.program_id(2)
        my_y = lax.axis_index("y")
        peer = _y_peer()
        rdma = pltpu.make_async_remote_copy(
            src_ref=ygw_ref, dst_ref=yrecv_ref, send_sem=s_sem,
            recv_sem=r_sem, device_id=peer, device_id_type=_MESH,
        )

        @pl.when((p == 0) & (r == 0) & (e == 0))
        def _():
            _neighbor_barrier(peer)
            rdma.start()

        @pl.when((p == 1) & (r == 0) & (e == 0))
        def _():
            rdma.wait()

        rs = pl.ds(r * CSY, CSY)

        @pl.when(p == 0)
        def _():
            pm = _onehot_rows(s_ref, r + n_r, e * CAP + my_y * CAPH, CAPH)
            contrib = jnp.dot(
                pm, yo_ref[0], preferred_element_type=jnp.float32
            ).astype(jnp.bfloat16)

            @pl.when(e == 0)
            def _():
                acc_ref[rs, :] = contrib

            @pl.when(e != 0)
            def _():
                acc_ref[rs, :] += contrib

        @pl.when(p == 1)
        def _():
            pr = _onehot_rows(
                s_ref, r + n_r, e * CAP + (1 - my_y) * CAPH, CAPH
            )
            contrib = jnp.dot(
                pr, yrecv_ref[e], preferred_element_type=jnp.float32
            ).astype(jnp.bfloat16)
            acc_ref[rs, :] += contrib

            @pl.when(e == E_LOCAL - 1)
            def _():
                o_ref[...] = acc_ref[rs, :]

    return pl.pallas_call(
        body,
        grid=(2, n_r, E_LOCAL),
        in_specs=[
            pl.BlockSpec(memory_space=pltpu.VMEM),
            pl.BlockSpec(memory_space=pltpu.VMEM),
            pl.BlockSpec((1, CAPH, D), lambda p, r, e: (e, 0, 0)),
        ],
        out_specs=(
            pl.BlockSpec((CSY, D), lambda p, r, e: (r, 0)),
            pl.BlockSpec(memory_space=pltpu.VMEM),
        ),
        out_shape=(
            jax.ShapeDtypeStruct((T, D), jnp.bfloat16),
            jax.ShapeDtypeStruct((E_LOCAL, CAPH, D), jnp.bfloat16),
        ),
        scratch_shapes=[
            pltpu.VMEM((T, D), jnp.bfloat16),
            pltpu.SemaphoreType.DMA,
            pltpu.SemaphoreType.DMA,
        ],
        compiler_params=pltpu.CompilerParams(
            collective_id=2, vmem_limit_bytes=_VMEM_LIMIT
        ),
    )(ygh, s_all, ygh)


def _scatter_mine_exchange(theirs_bf, s_all, ygh_own, ygh_recv):
    n_r = T // CS

    def body(pt_ref, s_ref, yo_ref, yr_ref, om_ref, xrecv_ref, s_sem, r_sem):
        r = pl.program_id(0)
        e = pl.program_id(1)
        my_y = lax.axis_index("y")
        peer = _x_peer()
        rdma = pltpu.make_async_remote_copy(
            src_ref=pt_ref, dst_ref=xrecv_ref, send_sem=s_sem,
            recv_sem=r_sem, device_id=peer, device_id_type=_MESH,
        )

        @pl.when((r == 0) & (e == 0))
        def _():
            _neighbor_barrier(peer)
            rdma.start()

        contrib = _scatter_contrib(s_ref, r, e, yo_ref, yr_ref, my_y)

        @pl.when(e == 0)
        def _():
            om_ref[...] = contrib.astype(jnp.bfloat16)

        @pl.when(e != 0)
        def _():
            om_ref[...] = om_ref[...] + contrib.astype(jnp.bfloat16)

        @pl.when((r == n_r - 1) & (e == E_LOCAL - 1))
        def _():
            rdma.wait()

    return pl.pallas_call(
        body,
        grid=(n_r, E_LOCAL),
        in_specs=[
            pl.BlockSpec(memory_space=pltpu.VMEM),
            pl.BlockSpec(memory_space=pltpu.VMEM),
            pl.BlockSpec((1, CAPH, D), lambda r, e: (e, 0, 0)),
            pl.BlockSpec((1, CAPH, D), lambda r, e: (e, 0, 0)),
        ],
        out_specs=(
            pl.BlockSpec((CS, D), lambda r, e: (r, 0)),
            pl.BlockSpec(memory_space=pl.ANY),
        ),
        out_shape=(
            jax.ShapeDtypeStruct((T, D), jnp.bfloat16),
            jax.ShapeDtypeStruct((T, D), jnp.bfloat16),
        ),
        scratch_shapes=[pltpu.SemaphoreType.DMA, pltpu.SemaphoreType.DMA],
        compiler_params=pltpu.CompilerParams(
            collective_id=1, vmem_limit_bytes=_VMEM_LIMIT
        ),
    )(theirs_bf, s_all, ygh_own, ygh_recv)


def _slot_ids(assign_all, e_base):
    oh = (
        assign_all[None, :] == jnp.arange(N_EXP, dtype=jnp.int32)[:, None]
    ).astype(jnp.int32)
    rank = (oh * jnp.cumsum(oh, axis=1)).sum(0) - 1
    loc = assign_all - e_base
    valid = (loc >= 0) & (loc < E_LOCAL) & (rank < CAP)
    return jnp.where(valid, loc * CAP + rank, -1).astype(jnp.int32)


def kernel(x, assign, W1, W2):
    my_x = lax.axis_index("x")
    x_bf = x.astype(jnp.bfloat16)
    a2d = assign.reshape(32, 128)

    x_other, a_other = _exchange(x_bf, a2d)
    assign_all = jnp.concatenate([assign, a_other.reshape(-1)])

    s_all = _slot_ids(assign_all, my_x * E_LOCAL)
    s2d = s_all.reshape(2 * T // CG, CG)

    xg = _gather_onehot(s2d[: T // CG], s2d[T // CG:], x_bf, x_other)
    ygh = _moe_ffn(xg, W1, W2)

    theirs, ygh_recv = _scatter_theirs_yexchange(
        ygh, s_all.reshape(2 * T // CSY, CSY)
    )
    mine, recv = _scatter_mine_exchange(
        theirs, s_all.reshape(2 * T // CS, CS), ygh, ygh_recv
    )
    return mine.astype(jnp.float32) + recv.astype(jnp.float32)


# device time: 882492 ns/iter; 3.8841x vs baseline; 1.0419x over previous
import jax
import jax.numpy as jnp
from jax import lax
from jax.experimental import pallas as pl
from jax.experimental.pallas import tpu as pltpu

T = 4096
D = 2048
F = 4096
E_LOCAL = 4
N_EXP = 8
CAP = 1280
CAPH = CAP // 2
BT = 320
FB = 1024
NF = F // FB
CG = 1024
CS = 1024
CSY = 512

_MESH = pl.DeviceIdType.MESH
_VMEM_LIMIT = 60 * 1024 * 1024


def _x_peer():
    return (1 - lax.axis_index("x"), lax.axis_index("y"))


def _y_peer():
    return (lax.axis_index("x"), 1 - lax.axis_index("y"))


def _neighbor_barrier(peer):
    bar = pltpu.get_barrier_semaphore()
    pl.semaphore_signal(bar, inc=1, device_id=peer, device_id_type=_MESH)
    pl.semaphore_wait(bar, 1)


def _exchange(x_bf, a2d):

    def body(x_ref, a_ref, xo_ref, ao_ref, xs_sem, xr_sem, as_sem, ar_sem):
        peer = _x_peer()
        _neighbor_barrier(peer)
        rx = pltpu.make_async_remote_copy(
            src_ref=x_ref, dst_ref=xo_ref, send_sem=xs_sem, recv_sem=xr_sem,
            device_id=peer, device_id_type=_MESH,
        )
        ra = pltpu.make_async_remote_copy(
            src_ref=a_ref, dst_ref=ao_ref, send_sem=as_sem, recv_sem=ar_sem,
            device_id=peer, device_id_type=_MESH,
        )
        rx.start()
        ra.start()
        rx.wait()
        ra.wait()

    return pl.pallas_call(
        body,
        out_shape=(
            jax.ShapeDtypeStruct((T, D), jnp.bfloat16),
            jax.ShapeDtypeStruct(a2d.shape, jnp.int32),
        ),
        in_specs=[
            pl.BlockSpec(memory_space=pltpu.VMEM),
            pl.BlockSpec(memory_space=pltpu.VMEM),
        ],
        out_specs=(
            pl.BlockSpec(memory_space=pltpu.VMEM),
            pl.BlockSpec(memory_space=pltpu.VMEM),
        ),
        scratch_shapes=[pltpu.SemaphoreType.DMA] * 4,
        compiler_params=pltpu.CompilerParams(
            collective_id=0, vmem_limit_bytes=_VMEM_LIMIT
        ),
    )(x_bf, a2d)


def _slot_iota(e):
    my_y = lax.axis_index("y")
    return (
        e * CAP + my_y * CAPH
        + lax.broadcasted_iota(jnp.int32, (CAPH, CG), 0)
    )


def _exchange_gather_mine(x_bf, a2d, s_mine):

    def body(x_ref, a_ref, sm_ref, xo_ref, ao_ref, o_ref,
             xs_sem, xr_sem, as_sem, ar_sem):
        e = pl.program_id(0)
        c = pl.program_id(1)
        peer = _x_peer()
        rx = pltpu.make_async_remote_copy(
            src_ref=x_ref, dst_ref=xo_ref, send_sem=xs_sem, recv_sem=xr_sem,
            device_id=peer, device_id_type=_MESH,
        )
        ra = pltpu.make_async_remote_copy(
            src_ref=a_ref, dst_ref=ao_ref, send_sem=as_sem, recv_sem=ar_sem,
            device_id=peer, device_id_type=_MESH,
        )

        @pl.when((e == 0) & (c == 0))
        def _():
            _neighbor_barrier(peer)
            rx.start()
            ra.start()

        pm = (_slot_iota(e) == sm_ref[c][None, :]).astype(jnp.bfloat16)
        x_chunk = x_ref[pl.ds(c * CG, CG), :]
        contrib = jnp.dot(pm, x_chunk, preferred_element_type=jnp.float32)

        @pl.when(c == 0)
        def _():
            o_ref[0] = contrib.astype(jnp.bfloat16)

        @pl.when(c != 0)
        def _():
            o_ref[0] = o_ref[0] + contrib.astype(jnp.bfloat16)

        @pl.when((e == E_LOCAL - 1) & (c == T // CG - 1))
        def _():
            rx.wait()
            ra.wait()

    return pl.pallas_call(
        body,
        grid=(E_LOCAL, T // CG),
        in_specs=[
            pl.BlockSpec(memory_space=pltpu.VMEM),
            pl.BlockSpec(memory_space=pltpu.VMEM),
            pl.BlockSpec(memory_space=pltpu.VMEM),
        ],
        out_specs=(
            pl.BlockSpec(memory_space=pltpu.VMEM),
            pl.BlockSpec(memory_space=pltpu.VMEM),
            pl.BlockSpec((1, CAPH, D), lambda e, c: (e, 0, 0)),
        ),
        out_shape=(
            jax.ShapeDtypeStruct((T, D), jnp.bfloat16),
            jax.ShapeDtypeStruct(a2d.shape, jnp.int32),
            jax.ShapeDtypeStruct((E_LOCAL, CAPH, D), jnp.bfloat16),
        ),
        scratch_shapes=[pltpu.SemaphoreType.DMA] * 4,
        compiler_params=pltpu.CompilerParams(
            collective_id=0, vmem_limit_bytes=_VMEM_LIMIT
        ),
    )(x_bf, a2d, s_mine)


def _gather_other(s_other, xgp, x_other):

    def body(so_ref, xgp_ref, xo_ref, o_ref):
        e = pl.program_id(0)
        c = pl.program_id(1)
        po = (_slot_iota(e) == so_ref[c][None, :]).astype(jnp.bfloat16)
        contrib = jnp.dot(po, xo_ref[...], preferred_element_type=jnp.float32)

        @pl.when(c == 0)
        def _():
            o_ref[0] = xgp_ref[0] + contrib.astype(jnp.bfloat16)

        @pl.when(c != 0)
        def _():
            o_ref[0] = o_ref[0] + contrib.astype(jnp.bfloat16)

    return pl.pallas_call(
        body,
        grid=(E_LOCAL, T // CG),
        in_specs=[
            pl.BlockSpec(memory_space=pltpu.VMEM),
            pl.BlockSpec((1, CAPH, D), lambda e, c: (e, 0, 0)),
            pl.BlockSpec((CG, D), lambda e, c: (c, 0)),
        ],
        out_specs=pl.BlockSpec((1, CAPH, D), lambda e, c: (e, 0, 0)),
        out_shape=jax.ShapeDtypeStruct((E_LOCAL, CAPH, D), jnp.bfloat16),
        compiler_params=pltpu.CompilerParams(vmem_limit_bytes=_VMEM_LIMIT),
    )(s_other, xgp, x_other)


def _moe_ffn(xg, w1, w2):

    def body(x_ref, w1_ref, w2_ref, o_ref, w1b, w2b, acc_ref):
        f = pl.program_id(1)
        t = pl.program_id(2)

        @pl.when(t == 0)
        def _():
            w1b[...] = w1_ref[0].astype(jnp.bfloat16)
            w2b[...] = w2_ref[0].astype(jnp.bfloat16)

        h = jnp.dot(x_ref[0], w1b[...], preferred_element_type=jnp.float32)
        h = jnp.maximum(h, 0.0).astype(jnp.bfloat16)
        p = jnp.dot(h, w2b[...], preferred_element_type=jnp.float32)
        sl = pl.ds(t * BT, BT)

        @pl.when(f == 0)
        def _():
            acc_ref[sl, :] = p

        @pl.when(f != 0)
        def _():
            acc_ref[sl, :] += p

        @pl.when(f == NF - 1)
        def _():
            o_ref[0] = acc_ref[sl, :].astype(jnp.bfloat16)

    return pl.pallas_call(
        body,
        grid=(E_LOCAL, NF, CAPH // BT),
        in_specs=[
            pl.BlockSpec((1, BT, D), lambda e, f, t: (e, t, 0)),
            pl.BlockSpec((1, D, FB), lambda e, f, t: (e, 0, f)),
            pl.BlockSpec((1, FB, D), lambda e, f, t: (e, f, 0)),
        ],
        out_specs=pl.BlockSpec((1, BT, D), lambda e, f, t: (e, t, 0)),
        out_shape=jax.ShapeDtypeStruct((E_LOCAL, CAPH, D), jnp.bfloat16),
        scratch_shapes=[
            pltpu.VMEM((D, FB), jnp.bfloat16),
            pltpu.VMEM((FB, D), jnp.bfloat16),
            pltpu.VMEM((CAPH, D), jnp.float32),
        ],
        compiler_params=pltpu.CompilerParams(vmem_limit_bytes=_VMEM_LIMIT),
    )(xg, w1, w2)


def _onehot_rows(s_ref, chunk, base, n_slots):
    s_col = s_ref[chunk][:, None]
    rows = s_ref.shape[1]
    rk = lax.broadcasted_iota(jnp.int32, (rows, n_slots), 1)
    return (s_col == base + rk).astype(jnp.bfloat16)


def _scatter_contrib(s_ref, chunk, e, yo_ref, yr_ref, my_y):
    p_own = _onehot_rows(s_ref, chunk, e * CAP + my_y * CAPH, CAPH)
    p_recv = _onehot_rows(s_ref, chunk, e * CAP + (1 - my_y) * CAPH, CAPH)
    return jnp.dot(
        p_own, yo_ref[0], preferred_element_type=jnp.float32
    ) + jnp.dot(p_recv, yr_ref[0], preferred_element_type=jnp.float32)


def _scatter_theirs_yexchange(ygh, s_all):
    n_r = T // CSY

    def body(ygw_ref, s_ref, yo_ref, o_ref, yrecv_ref, acc_ref, s_sem, r_sem):
        p = pl.program_id(0)
        r = pl.program_id(1)
        e = pl.program_id(2)
        my_y = lax.axis_index("y")
        peer = _y_peer()
        rdma = pltpu.make_async_remote_copy(
            src_ref=ygw_ref, dst_ref=yrecv_ref, send_sem=s_sem,
            recv_sem=r_sem, device_id=peer, device_id_type=_MESH,
        )

        @pl.when((p == 0) & (r == 0) & (e == 0))
        def _():
            _neighbor_barrier(peer)
            rdma.start()

        @pl.when((p == 1) & (r == 0) & (e == 0))
        def _():
            rdma.wait()

        rs = pl.ds(r * CSY, CSY)

        @pl.when(p == 0)
        def _():
            pm = _onehot_rows(s_ref, r + n_r, e * CAP + my_y * CAPH, CAPH)
            contrib = jnp.dot(
                pm, yo_ref[0], preferred_element_type=jnp.float32
            ).astype(jnp.bfloat16)

            @pl.when(e == 0)
            def _():
                acc_ref[rs, :] = contrib

            @pl.when(e != 0)
            def _():
                acc_ref[rs, :] += contrib

        @pl.when(p == 1)
        def _():
            pr = _onehot_rows(
                s_ref, r + n_r, e * CAP + (1 - my_y) * CAPH, CAPH
            )
            contrib = jnp.dot(
                pr, yrecv_ref[e], preferred_element_type=jnp.float32
            ).astype(jnp.bfloat16)
            acc_ref[rs, :] += contrib

            @pl.when(e == E_LOCAL - 1)
            def _():
                o_ref[...] = acc_ref[rs, :]

    return pl.pallas_call(
        body,
        grid=(2, n_r, E_LOCAL),
        in_specs=[
            pl.BlockSpec(memory_space=pltpu.VMEM),
            pl.BlockSpec(memory_space=pltpu.VMEM),
            pl.BlockSpec((1, CAPH, D), lambda p, r, e: (e, 0, 0)),
        ],
        out_specs=(
            pl.BlockSpec((CSY, D), lambda p, r, e: (r, 0)),
            pl.BlockSpec(memory_space=pltpu.VMEM),
        ),
        out_shape=(
            jax.ShapeDtypeStruct((T, D), jnp.bfloat16),
            jax.ShapeDtypeStruct((E_LOCAL, CAPH, D), jnp.bfloat16),
        ),
        scratch_shapes=[
            pltpu.VMEM((T, D), jnp.bfloat16),
            pltpu.SemaphoreType.DMA,
            pltpu.SemaphoreType.DMA,
        ],
        compiler_params=pltpu.CompilerParams(
            collective_id=2, vmem_limit_bytes=_VMEM_LIMIT
        ),
    )(ygh, s_all, ygh)


def _scatter_mine_exchange(theirs_bf, s_all, ygh_own, ygh_recv):
    n_r = T // CS

    def body(pt_ref, s_ref, yo_ref, yr_ref, om_ref, xrecv_ref, s_sem, r_sem):
        r = pl.program_id(0)
        e = pl.program_id(1)
        my_y = lax.axis_index("y")
        peer = _x_peer()
        rdma = pltpu.make_async_remote_copy(
            src_ref=pt_ref, dst_ref=xrecv_ref, send_sem=s_sem,
            recv_sem=r_sem, device_id=peer, device_id_type=_MESH,
        )

        @pl.when((r == 0) & (e == 0))
        def _():
            _neighbor_barrier(peer)
            rdma.start()

        contrib = _scatter_contrib(s_ref, r, e, yo_ref, yr_ref, my_y)

        @pl.when(e == 0)
        def _():
            om_ref[...] = contrib.astype(jnp.bfloat16)

        @pl.when(e != 0)
        def _():
            om_ref[...] = om_ref[...] + contrib.astype(jnp.bfloat16)

        @pl.when((r == n_r - 1) & (e == E_LOCAL - 1))
        def _():
            rdma.wait()

    return pl.pallas_call(
        body,
        grid=(n_r, E_LOCAL),
        in_specs=[
            pl.BlockSpec(memory_space=pltpu.VMEM),
            pl.BlockSpec(memory_space=pltpu.VMEM),
            pl.BlockSpec((1, CAPH, D), lambda r, e: (e, 0, 0)),
            pl.BlockSpec((1, CAPH, D), lambda r, e: (e, 0, 0)),
        ],
        out_specs=(
            pl.BlockSpec((CS, D), lambda r, e: (r, 0)),
            pl.BlockSpec(memory_space=pl.ANY),
        ),
        out_shape=(
            jax.ShapeDtypeStruct((T, D), jnp.bfloat16),
            jax.ShapeDtypeStruct((T, D), jnp.bfloat16),
        ),
        scratch_shapes=[pltpu.SemaphoreType.DMA, pltpu.SemaphoreType.DMA],
        compiler_params=pltpu.CompilerParams(
            collective_id=1, vmem_limit_bytes=_VMEM_LIMIT
        ),
    )(theirs_bf, s_all, ygh_own, ygh_recv)


def _slot_ids(assign_vec, e_base, base_counts=None):
    oh = (
        assign_vec[None, :] == jnp.arange(N_EXP, dtype=jnp.int32)[:, None]
    ).astype(jnp.int32)
    rank = (oh * jnp.cumsum(oh, axis=1)).sum(0) - 1
    if base_counts is not None:
        rank = rank + (oh * base_counts[:, None]).sum(0)
    loc = assign_vec - e_base
    valid = (loc >= 0) & (loc < E_LOCAL) & (rank < CAP)
    s = jnp.where(valid, loc * CAP + rank, -1).astype(jnp.int32)
    return s, oh.sum(axis=1)


def kernel(x, assign, W1, W2):
    my_x = lax.axis_index("x")
    e_base = my_x * E_LOCAL
    x_bf = x.astype(jnp.bfloat16)
    a2d = assign.reshape(32, 128)

    s_mine, cnt_mine = _slot_ids(assign, e_base)
    x_other, a_other, xgp = _exchange_gather_mine(
        x_bf, a2d, s_mine.reshape(T // CG, CG)
    )
    s_other, _ = _slot_ids(a_other.reshape(-1), e_base, base_counts=cnt_mine)

    xg = _gather_other(s_other.reshape(T // CG, CG), xgp, x_other)
    ygh = _moe_ffn(xg, W1, W2)

    s_all = jnp.concatenate([s_mine, s_other])
    theirs, ygh_recv = _scatter_theirs_yexchange(
        ygh, s_all.reshape(2 * T // CSY, CSY)
    )
    mine, recv = _scatter_mine_exchange(
        theirs, s_all.reshape(2 * T // CS, CS), ygh, ygh_recv
    )
    return mine.astype(jnp.float32) + recv.astype(jnp.float32)
